# baseline (speedup 1.0000x reference)
import functools

import jax
import jax.numpy as jnp
from jax import lax
from jax.experimental import pallas as pl
from jax.experimental.pallas import tpu as pltpu
from jax.experimental.pallas import tpu_sc as plsc

VOCAB = 1000
D = 1000
VP = 1024
B, T = 4096, 20
GD = 40
NG = D // GD
L = 16

_info = plsc.get_sparse_core_info()
NC, NS = _info.num_cores, _info.num_subcores
NW = NC * NS
BW = B // NW
NBG = BW // L

_W_BYTES = GD * BW * 4
_T_BYTES = GD * VP * 4

_mesh = plsc.VectorSubcoreMesh(core_axis_name="c", subcore_axis_name="s")


@functools.partial(
    pl.kernel,
    mesh=_mesh,
    out_type=jax.ShapeDtypeStruct((T, D, B), jnp.float32),
    scratch_types=[
        pltpu.VMEM((T, BW), jnp.int32),
        [pltpu.VMEM((GD * VP,), jnp.float32) for _ in range(2)],
        [pltpu.VMEM((GD, BW), jnp.float32) for _ in range(2)],
        [pltpu.SemaphoreType.DMA] * 2,
        [pltpu.SemaphoreType.DMA] * 2,
    ],
    compiler_params=pltpu.CompilerParams(
        use_tc_tiling_on_sc=True, needs_layout_passes=False),
)
def _gather_kernel(idxT_hbm, ttf_hbm, out_hbm, idx_v, tts, slabs, tsem, wsem):
    wid = lax.axis_index("s") * NC + lax.axis_index("c")
    b0 = wid * BW

    def t_load(g, a):
        return pltpu.make_async_copy(
            ttf_hbm.at[pl.ds(g * (GD * VP), GD * VP)], tts[a], tsem[a])

    def w_copy(t, g, sl):
        return pltpu.make_async_copy(
            slabs[sl], out_hbm.at[t, pl.ds(g * GD, GD), pl.ds(b0, BW)], wsem[sl])

    def build_and_write(t, g, a, sl):
        w_copy(0, 0, sl).wait()
        ivs = [idx_v[t, pl.ds(L * k, L)] for k in range(NBG)]

        def per_d(dloc, carry):
            base = dloc * VP
            for k in range(NBG):
                x = plsc.load_gather(tts[a], [ivs[k] + base])
                slabs[sl][dloc, pl.ds(L * k, L)] = x
            return carry

        lax.fori_loop(0, GD, per_d, 0)
        w_copy(t, g, sl).start()

    def twenty_ts(g, a):
        def t_pair(p, carry):
            build_and_write(2 * p, g, a, 0)
            build_and_write(2 * p + 1, g, a, 1)
            return carry
        lax.fori_loop(0, T // 2, t_pair, 0)

    w_copy(0, 0, 0).start()
    w_copy(1, 0, 1).start()

    pltpu.sync_copy(idxT_hbm.at[:, pl.ds(b0, BW)], idx_v)
    t_load(0, 0).start()
    t_load(1, 1).start()

    def dgroup(r, carry):
        g = 2 * r
        t_load(0, 0).wait()
        twenty_ts(g, 0)
        t_load(g + 2, 0).start()
        t_load(0, 1).wait()
        twenty_ts(g + 1, 1)

        @pl.when(r < NG // 2 - 1)
        def _():
            t_load(g + 3, 1).start()
        return carry

    lax.fori_loop(0, NG // 2, dgroup, 0)

    t_load(0, 0).wait()
    twenty_ts(NG - 1, 0)

    w_copy(0, 0, 0).wait()
    w_copy(0, 0, 1).wait()


def kernel(index, table):
    idxT = index.T.astype(jnp.int32)
    ttf = jnp.pad(table.T, ((0, 0), (0, VP - VOCAB))).reshape(-1)
    out_phys = _gather_kernel(idxT, ttf)
    return jnp.transpose(out_phys, (2, 0, 1))


# speedup vs baseline: 2.7874x; 2.7874x over previous
import functools

import jax
import jax.numpy as jnp
from jax import lax
from jax.experimental import pallas as pl
from jax.experimental.pallas import tpu as pltpu
from jax.experimental.pallas import tpu_sc as plsc

VOCAB = 1000
D = 1000
VP = 1024
B, T = 4096, 20
GD = 40
NG = D // GD
L = 16

_info = plsc.get_sparse_core_info()
NC, NS = _info.num_cores, _info.num_subcores
NW = NC * NS
BW = B // NW
NBG = BW // L

_W_BYTES = GD * BW * 4
_T_BYTES = GD * VP * 4

_mesh = plsc.VectorSubcoreMesh(core_axis_name="c", subcore_axis_name="s")


@functools.partial(
    pl.kernel,
    mesh=_mesh,
    out_type=jax.ShapeDtypeStruct((T, D, B), jnp.float32),
    scratch_types=[
        pltpu.VMEM((T, BW), jnp.int32),
        [pltpu.VMEM((GD * VP,), jnp.float32) for _ in range(2)],
        [pltpu.VMEM((GD, BW), jnp.float32) for _ in range(2)],
        [pltpu.SemaphoreType.DMA] * 2,
        [pltpu.SemaphoreType.DMA] * 2,
    ],
    compiler_params=pltpu.CompilerParams(
        use_tc_tiling_on_sc=True, needs_layout_passes=False),
)
def _gather_kernel(idxT_hbm, ttf_hbm, out_hbm, idx_v, tts, slabs, tsem, wsem):
    wid = lax.axis_index("s") * NC + lax.axis_index("c")
    b0 = wid * BW

    def t_load(g, a):
        return pltpu.make_async_copy(
            ttf_hbm.at[pl.ds(g * (GD * VP), GD * VP)], tts[a], tsem[a])

    def w_copy(t, g, sl):
        return pltpu.make_async_copy(
            slabs[sl], out_hbm.at[t, pl.ds(g * GD, GD), pl.ds(b0, BW)], wsem[sl])

    def build_and_write(t, g, a, sl):
        w_copy(0, 0, sl).wait()
        ivs = [idx_v[t, pl.ds(L * k, L)] for k in range(NBG)]

        def per_d(dloc, carry):
            base = dloc * VP
            xs = [plsc.load_gather(tts[a], [ivs[k] + base]) for k in range(NBG)]
            for k in range(NBG):
                slabs[sl][dloc, pl.ds(L * k, L)] = xs[k]
            return carry

        lax.fori_loop(0, GD, per_d, 0, unroll=4)

        w_copy(t, g, sl).start()

    def twenty_ts(g, a):
        def t_pair(p, carry):
            build_and_write(2 * p, g, a, 0)
            build_and_write(2 * p + 1, g, a, 1)
            return carry
        lax.fori_loop(0, T // 2, t_pair, 0)

    w_copy(0, 0, 0).start()
    w_copy(1, 0, 1).start()

    pltpu.sync_copy(idxT_hbm.at[:, pl.ds(b0, BW)], idx_v)
    t_load(0, 0).start()
    t_load(1, 1).start()

    def dgroup(r, carry):
        g = 2 * r
        t_load(0, 0).wait()
        twenty_ts(g, 0)
        t_load(g + 2, 0).start()
        t_load(0, 1).wait()
        twenty_ts(g + 1, 1)

        @pl.when(r < NG // 2 - 1)
        def _():
            t_load(g + 3, 1).start()
        return carry

    lax.fori_loop(0, NG // 2, dgroup, 0)

    t_load(0, 0).wait()
    twenty_ts(NG - 1, 0)

    w_copy(0, 0, 0).wait()
    w_copy(0, 0, 1).wait()


def kernel(index, table):
    idxT = index.T.astype(jnp.int32)
    ttf = jnp.pad(table.T, ((0, 0), (0, VP - VOCAB))).reshape(-1)
    out_phys = _gather_kernel(idxT, ttf)
    return jnp.transpose(out_phys, (2, 0, 1))


# speedup vs baseline: 2.8041x; 1.0060x over previous
import functools

import jax
import jax.numpy as jnp
from jax import lax
from jax.experimental import pallas as pl
from jax.experimental.pallas import tpu as pltpu
from jax.experimental.pallas import tpu_sc as plsc

VOCAB = 1000
D = 1000
VP = 1024
B, T = 4096, 20
GD = 40
NG = D // GD
L = 16

_info = plsc.get_sparse_core_info()
NC, NS = _info.num_cores, _info.num_subcores
NW = NC * NS
BW = B // NW
NBG = BW // L

_W_BYTES = GD * BW * 4
_T_BYTES = GD * VP * 4

_mesh = plsc.VectorSubcoreMesh(core_axis_name="c", subcore_axis_name="s")


@functools.partial(
    pl.kernel,
    mesh=_mesh,
    out_type=jax.ShapeDtypeStruct((T, D, B), jnp.float32),
    scratch_types=[
        pltpu.VMEM((T, BW), jnp.int32),
        [pltpu.VMEM((GD * VP,), jnp.float32) for _ in range(2)],
        [pltpu.VMEM((GD, BW), jnp.float32) for _ in range(2)],
        [pltpu.SemaphoreType.DMA] * 2,
        [pltpu.SemaphoreType.DMA] * 2,
    ],
    compiler_params=pltpu.CompilerParams(
        use_tc_tiling_on_sc=True, needs_layout_passes=False),
)
def _gather_kernel(idxT_hbm, ttf_hbm, out_hbm, idx_v, tts, slabs, tsem, wsem):
    wid = lax.axis_index("s") * NC + lax.axis_index("c")
    b0 = wid * BW

    def t_load(g, a):
        return pltpu.make_async_copy(
            ttf_hbm.at[pl.ds(g * (GD * VP), GD * VP)], tts[a], tsem[a])

    def w_copy(t, g, sl):
        return pltpu.make_async_copy(
            slabs[sl], out_hbm.at[t, pl.ds(g * GD, GD), pl.ds(b0, BW)], wsem[sl])

    def build_and_write(t, g, a, sl):
        w_copy(0, 0, sl).wait()
        ivs = [idx_v[t, pl.ds(L * k, L)] for k in range(NBG)]

        def per_d(dloc, carry):
            base = dloc * VP
            xs = [plsc.load_gather(tts[a], [ivs[k] + base]) for k in range(NBG)]
            for k in range(NBG):
                slabs[sl][dloc, pl.ds(L * k, L)] = xs[k]
            return carry

        lax.fori_loop(0, GD, per_d, 0, unroll=8)

        w_copy(t, g, sl).start()

    def twenty_ts(g, a):
        def t_pair(p, carry):
            build_and_write(2 * p, g, a, 0)
            build_and_write(2 * p + 1, g, a, 1)
            return carry
        lax.fori_loop(0, T // 2, t_pair, 0)

    w_copy(0, 0, 0).start()
    w_copy(1, 0, 1).start()

    pltpu.sync_copy(idxT_hbm.at[:, pl.ds(b0, BW)], idx_v)
    t_load(0, 0).start()
    t_load(1, 1).start()

    def dgroup(r, carry):
        g = 2 * r
        t_load(0, 0).wait()
        twenty_ts(g, 0)
        t_load(g + 2, 0).start()
        t_load(0, 1).wait()
        twenty_ts(g + 1, 1)

        @pl.when(r < NG // 2 - 1)
        def _():
            t_load(g + 3, 1).start()
        return carry

    lax.fori_loop(0, NG // 2, dgroup, 0)

    t_load(0, 0).wait()
    twenty_ts(NG - 1, 0)

    w_copy(0, 0, 0).wait()
    w_copy(0, 0, 1).wait()


def kernel(index, table):
    idxT = index.T.astype(jnp.int32)
    ttf = jnp.pad(table.T, ((0, 0), (0, VP - VOCAB))).reshape(-1)
    out_phys = _gather_kernel(idxT, ttf)
    return jnp.transpose(out_phys, (2, 0, 1))


# speedup vs baseline: 2.8348x; 1.0109x over previous
import functools

import jax
import jax.numpy as jnp
from jax import lax
from jax.experimental import pallas as pl
from jax.experimental.pallas import tpu as pltpu
from jax.experimental.pallas import tpu_sc as plsc

VOCAB = 1000
D = 1000
VP = 1024
B, T = 4096, 20
GD = 40
NG = D // GD
L = 16

_info = plsc.get_sparse_core_info()
NC, NS = _info.num_cores, _info.num_subcores
NW = NC * NS
BW = B // NW
NBG = BW // L

_W_BYTES = GD * BW * 4
_T_BYTES = GD * VP * 4

_mesh = plsc.VectorSubcoreMesh(core_axis_name="c", subcore_axis_name="s")


@functools.partial(
    pl.kernel,
    mesh=_mesh,
    out_type=jax.ShapeDtypeStruct((T, D, B), jnp.float32),
    scratch_types=[
        pltpu.VMEM((T, BW), jnp.int32),
        [pltpu.VMEM((GD * VP,), jnp.float32) for _ in range(2)],
        [pltpu.VMEM((GD, BW), jnp.float32) for _ in range(2)],
        [pltpu.SemaphoreType.DMA] * 2,
        [pltpu.SemaphoreType.DMA] * 2,
    ],
    compiler_params=pltpu.CompilerParams(
        use_tc_tiling_on_sc=True, needs_layout_passes=False),
)
def _gather_kernel(idxT_hbm, ttf_hbm, out_hbm, idx_v, tts, slabs, tsem, wsem):
    wid = lax.axis_index("s") * NC + lax.axis_index("c")
    b0 = wid * BW

    def t_load(g, a):
        return pltpu.make_async_copy(
            ttf_hbm.at[pl.ds(g * (GD * VP), GD * VP)], tts[a], tsem[a])

    def w_copy(t, g, sl):
        return pltpu.make_async_copy(
            slabs[sl], out_hbm.at[t, pl.ds(g * GD, GD), pl.ds(b0, BW)], wsem[sl])

    def build_and_write(t, g, a, sl):
        w_copy(0, 0, sl).wait()
        ivs = [idx_v[t, pl.ds(L * k, L)] for k in range(NBG)]

        def per_d(dloc, carry):
            row = tts[a].at[pl.ds(dloc * VP, VP)]
            xs = [plsc.load_gather(row, [ivs[k]]) for k in range(NBG)]
            for k in range(NBG):
                slabs[sl][dloc, pl.ds(L * k, L)] = xs[k]
            return carry

        lax.fori_loop(0, GD, per_d, 0, unroll=8)

        w_copy(t, g, sl).start()

    def twenty_ts(g, a):
        def t_pair(p, carry):
            build_and_write(2 * p, g, a, 0)
            build_and_write(2 * p + 1, g, a, 1)
            return carry
        lax.fori_loop(0, T // 2, t_pair, 0)

    w_copy(0, 0, 0).start()
    w_copy(1, 0, 1).start()

    pltpu.sync_copy(idxT_hbm.at[:, pl.ds(b0, BW)], idx_v)
    t_load(0, 0).start()
    t_load(1, 1).start()

    def dgroup(r, carry):
        g = 2 * r
        t_load(0, 0).wait()
        twenty_ts(g, 0)
        t_load(g + 2, 0).start()
        t_load(0, 1).wait()
        twenty_ts(g + 1, 1)

        @pl.when(r < NG // 2 - 1)
        def _():
            t_load(g + 3, 1).start()
        return carry

    lax.fori_loop(0, NG // 2, dgroup, 0)

    t_load(0, 0).wait()
    twenty_ts(NG - 1, 0)

    w_copy(0, 0, 0).wait()
    w_copy(0, 0, 1).wait()


def kernel(index, table):
    idxT = index.T.astype(jnp.int32)
    ttf = jnp.pad(table.T, ((0, 0), (0, VP - VOCAB))).reshape(-1)
    out_phys = _gather_kernel(idxT, ttf)
    return jnp.transpose(out_phys, (2, 0, 1))
